# baseline (speedup 1.0000x reference)
import functools

import jax
import jax.numpy as jnp
from jax import lax
from jax.experimental import pallas as pl
from jax.experimental.pallas import tpu as pltpu
from jax.experimental.pallas import tpu_sc as plsc

_B = 64
_K = 1024
_D = 32000
_KBLK = 256
_DBLK = 3200
_NK = _K // _KBLK
_ND = _D // _DBLK


def _dist_body(l_ref, c_ref, lm_ref, la_ref, lsq_ref, val_cur, lsq_cur):
    k = pl.program_id(0)
    d = pl.program_id(1)
    L = l_ref[:, pl.ds(d * _DBLK, _DBLK)]
    C = c_ref[...]

    l_hi = L.astype(jnp.bfloat16)
    c_hi = C.astype(jnp.bfloat16)
    c_hi32 = c_hi.astype(jnp.float32)
    t = C - c_hi32
    P = c_hi32 * (C + t)
    p_hi = P.astype(jnp.bfloat16)
    p_lo = (P - p_hi.astype(jnp.float32)).astype(jnp.bfloat16)

    ones8 = jnp.ones((8, _DBLK), jnp.bfloat16)
    dims = (((1,), (1,)), ((), ()))
    dot = lax.dot_general(l_hi, c_hi, dims, preferred_element_type=jnp.float32)
    csq8 = lax.dot_general(ones8, p_hi, dims, preferred_element_type=jnp.float32)
    csq8 = csq8 + lax.dot_general(ones8, p_lo, dims, preferred_element_type=jnp.float32)
    part = csq8[0:1, :] - 2.0 * dot

    val = part + jnp.where(d == 0, 0.0, val_cur[...])
    val_cur[...] = val

    lsq = jnp.sum(L * L, axis=1, keepdims=True)
    lsq_full = lsq + jnp.where(d == 0, 0.0, lsq_cur[...])
    lsq_cur[...] = lsq_full

    lmin = jnp.min(val, axis=1, keepdims=True)
    iota = lax.broadcasted_iota(jnp.int32, val.shape, 1)
    larg = jnp.min(jnp.where(val == lmin, iota, _KBLK),
                   axis=1, keepdims=True) + k * _KBLK
    lm_ref[...] = lmin[None]
    la_ref[...] = larg[None]
    lsq_ref[...] = lsq_full


def _merge_body(lm_ref, la_ref, lsq_ref, idx_ref, dist_ref):
    lm = lm_ref[...]
    la = la_ref[...]
    g = jnp.min(lm, axis=0, keepdims=True)
    cand = jnp.where(lm == g, la, _K)
    idx_ref[...] = jnp.min(cand, axis=0)
    dist_ref[...] = jnp.sqrt(jnp.maximum(g[0] + lsq_ref[...], 0.0))


def _nearest(latents_flat, codebook_flat):
    lm, la, lsq = pl.pallas_call(
        _dist_body,
        grid=(_NK, _ND),
        in_specs=[
            pl.BlockSpec((_B, _D), lambda k, d: (0, 0)),
            pl.BlockSpec((_KBLK, _DBLK), lambda k, d: (k, d)),
        ],
        out_specs=[
            pl.BlockSpec((1, _B, 1), lambda k, d: (k, 0, 0)),
            pl.BlockSpec((1, _B, 1), lambda k, d: (k, 0, 0)),
            pl.BlockSpec((_B, 1), lambda k, d: (0, 0)),
        ],
        out_shape=[
            jax.ShapeDtypeStruct((_NK, _B, 1), jnp.float32),
            jax.ShapeDtypeStruct((_NK, _B, 1), jnp.int32),
            jax.ShapeDtypeStruct((_B, 1), jnp.float32),
        ],
        scratch_shapes=[
            pltpu.VMEM((_B, _KBLK), jnp.float32),
            pltpu.VMEM((_B, 1), jnp.float32),
        ],
    )(latents_flat, codebook_flat)

    idx, dist = pl.pallas_call(
        _merge_body,
        out_shape=[
            jax.ShapeDtypeStruct((_B, 1), jnp.int32),
            jax.ShapeDtypeStruct((_B, 1), jnp.float32),
        ],
    )(lm, la, lsq)
    return idx[:, 0], dist[:, 0]


def _sc_gather(codebook_flat, indices):
    info = plsc.get_sparse_core_info()
    nw = info.num_cores * info.num_subcores
    bpw = _B // nw
    idx2d = indices.reshape(nw, bpw)
    mesh = plsc.VectorSubcoreMesh(core_axis_name="c", subcore_axis_name="s")

    @functools.partial(
        pl.kernel,
        mesh=mesh,
        out_type=jax.ShapeDtypeStruct((_B, _D), jnp.float32),
        scratch_types=[
            pltpu.VMEM((bpw,), jnp.int32),
            pltpu.VMEM((bpw, _D), jnp.float32),
            pltpu.SemaphoreType.DMA,
        ],
    )
    def gather(table_hbm, idx_hbm, out_hbm, idx_v, rows_v, sem):
        wid = lax.axis_index("s") * info.num_cores + lax.axis_index("c")
        pltpu.sync_copy(idx_hbm.at[wid], idx_v)
        pltpu.async_copy(table_hbm.at[idx_v], rows_v, sem).wait()
        pltpu.sync_copy(rows_v, out_hbm.at[pl.ds(wid * bpw, bpw)])

    return gather(codebook_flat, idx2d)


def kernel(latents, codebook):
    latents_flat = latents.reshape(_B, _D)
    codebook_flat = codebook.reshape(_K, _D)
    indices, min_distances = _nearest(latents_flat, codebook_flat)
    quantized = _sc_gather(codebook_flat, indices)
    quantized = quantized.reshape((_B,) + codebook.shape[1:])
    return (indices, quantized, min_distances)


# speedup vs baseline: 1.0073x; 1.0073x over previous
import functools

import jax
import jax.numpy as jnp
from jax import lax
from jax.experimental import pallas as pl
from jax.experimental.pallas import tpu as pltpu
from jax.experimental.pallas import tpu_sc as plsc

_B = 64
_K = 1024
_D = 32000
_KBLK = 256
_DBLK = 3200
_NK = _K // _KBLK
_ND = _D // _DBLK


def _dist_body(l_ref, c_ref, vp_ref, lq_ref):
    d = pl.program_id(1)
    L = l_ref[:, pl.ds(d * _DBLK, _DBLK)]
    C = c_ref[...]

    l_hi = L.astype(jnp.bfloat16)
    c_hi = C.astype(jnp.bfloat16)
    c_hi32 = c_hi.astype(jnp.float32)
    t = C - c_hi32
    P = c_hi32 * (C + t)
    p_hi = P.astype(jnp.bfloat16)
    p_lo = (P - p_hi.astype(jnp.float32)).astype(jnp.bfloat16)

    ones8 = jnp.ones((8, _DBLK), jnp.bfloat16)
    dims = (((1,), (1,)), ((), ()))
    dot = lax.dot_general(l_hi, c_hi, dims, preferred_element_type=jnp.float32)
    csq8 = lax.dot_general(ones8, p_hi, dims, preferred_element_type=jnp.float32)
    csq8 = csq8 + lax.dot_general(ones8, p_lo, dims, preferred_element_type=jnp.float32)
    vp_ref[...] = (csq8[0:1, :] - 2.0 * dot)[None]
    lq_ref[...] = jnp.sum(L * L, axis=1, keepdims=True)[None]


def _merge_body(vp_ref, lq_ref, idx_ref, dist_ref):
    vp = vp_ref[...]
    val = vp.reshape(_NK, _ND, _B, _KBLK).sum(axis=1)
    lsq = jnp.sum(lq_ref[...], axis=0)
    lmin = jnp.min(val, axis=2, keepdims=True)
    iota = lax.broadcasted_iota(jnp.int32, val.shape, 2)
    koff = lax.broadcasted_iota(jnp.int32, val.shape, 0) * _KBLK
    larg = jnp.min(jnp.where(val == lmin, iota + koff, _K),
                   axis=2, keepdims=True)
    g = jnp.min(lmin, axis=0, keepdims=True)
    cand = jnp.where(lmin == g, larg, _K)
    idx_ref[...] = jnp.min(cand, axis=0)
    dist_ref[...] = jnp.sqrt(jnp.maximum(g[0] + lsq, 0.0))


def _nearest(latents_flat, codebook_flat):
    vp, lq = pl.pallas_call(
        _dist_body,
        grid=(_NK, _ND),
        in_specs=[
            pl.BlockSpec((_B, _D), lambda k, d: (0, 0)),
            pl.BlockSpec((_KBLK, _DBLK), lambda k, d: (k, d)),
        ],
        out_specs=[
            pl.BlockSpec((1, _B, _KBLK), lambda k, d: (k * _ND + d, 0, 0)),
            pl.BlockSpec((1, _B, 1), lambda k, d: (d, 0, 0)),
        ],
        out_shape=[
            jax.ShapeDtypeStruct((_NK * _ND, _B, _KBLK), jnp.float32),
            jax.ShapeDtypeStruct((_ND, _B, 1), jnp.float32),
        ],
    )(latents_flat, codebook_flat)

    idx, dist = pl.pallas_call(
        _merge_body,
        out_shape=[
            jax.ShapeDtypeStruct((_B, 1), jnp.int32),
            jax.ShapeDtypeStruct((_B, 1), jnp.float32),
        ],
    )(vp, lq)
    return idx[:, 0], dist[:, 0]


def _sc_gather(codebook_flat, indices):
    info = plsc.get_sparse_core_info()
    nw = info.num_cores * info.num_subcores
    bpw = _B // nw
    idx2d = indices.reshape(nw, bpw)
    mesh = plsc.VectorSubcoreMesh(core_axis_name="c", subcore_axis_name="s")

    @functools.partial(
        pl.kernel,
        mesh=mesh,
        out_type=jax.ShapeDtypeStruct((_B, _D), jnp.float32),
        scratch_types=[
            pltpu.VMEM((bpw,), jnp.int32),
            pltpu.VMEM((bpw, _D), jnp.float32),
            pltpu.SemaphoreType.DMA,
        ],
    )
    def gather(table_hbm, idx_hbm, out_hbm, idx_v, rows_v, sem):
        wid = lax.axis_index("s") * info.num_cores + lax.axis_index("c")
        pltpu.sync_copy(idx_hbm.at[wid], idx_v)
        pltpu.async_copy(table_hbm.at[idx_v], rows_v, sem).wait()
        pltpu.sync_copy(rows_v, out_hbm.at[pl.ds(wid * bpw, bpw)])

    return gather(codebook_flat, idx2d)


def kernel(latents, codebook):
    latents_flat = latents.reshape(_B, _D)
    codebook_flat = codebook.reshape(_K, _D)
    indices, min_distances = _nearest(latents_flat, codebook_flat)
    quantized = _sc_gather(codebook_flat, indices)
    quantized = quantized.reshape((_B,) + codebook.shape[1:])
    return (indices, quantized, min_distances)


# speedup vs baseline: 1.0640x; 1.0563x over previous
import functools

import jax
import jax.numpy as jnp
from jax import lax
from jax.experimental import pallas as pl
from jax.experimental.pallas import tpu as pltpu
from jax.experimental.pallas import tpu_sc as plsc

_B = 64
_K = 1024
_D = 32000
_KBLK = 256
_DBLK = 6400
_NK = _K // _KBLK
_ND = _D // _DBLK


def _dist_body(l_ref, c_ref, idx_ref, dist_ref, val_acc, lsq_acc):
    d = pl.program_id(0)
    k = pl.program_id(1)
    L = l_ref[...]
    C = c_ref[...]

    l_hi = L.astype(jnp.bfloat16)
    c_hi = C.astype(jnp.bfloat16)
    c_hi32 = c_hi.astype(jnp.float32)
    t = C - c_hi32
    P = c_hi32 * (C + t)
    p_hi = P.astype(jnp.bfloat16)
    p_lo = (P - p_hi.astype(jnp.float32)).astype(jnp.bfloat16)

    ones8 = jnp.ones((8, _DBLK), jnp.bfloat16)
    dims = (((1,), (1,)), ((), ()))
    dot = lax.dot_general(l_hi, c_hi, dims, preferred_element_type=jnp.float32)
    csq8 = lax.dot_general(ones8, p_hi, dims, preferred_element_type=jnp.float32)
    csq8 = csq8 + lax.dot_general(ones8, p_lo, dims, preferred_element_type=jnp.float32)
    part = csq8[0:1, :] - 2.0 * dot

    @pl.when(d == 0)
    def _():
        val_acc[k] = part

    @pl.when(d > 0)
    def _():
        val_acc[k] += part

    @pl.when(k == 0)
    def _():
        lsq = jnp.sum(L * L, axis=1, keepdims=True)

        @pl.when(d == 0)
        def _():
            lsq_acc[...] = lsq

        @pl.when(d > 0)
        def _():
            lsq_acc[...] += lsq

    @pl.when(d == _ND - 1)
    def _():
        val = val_acc[k]
        lmin = jnp.min(val, axis=1, keepdims=True)
        iota = lax.broadcasted_iota(jnp.int32, val.shape, 1)
        larg = jnp.min(jnp.where(val == lmin, iota, _KBLK),
                       axis=1, keepdims=True) + k * _KBLK

        @pl.when(k == 0)
        def _():
            dist_ref[...] = lmin
            idx_ref[...] = larg

        @pl.when(k > 0)
        def _():
            better = lmin < dist_ref[...]
            dist_ref[...] = jnp.where(better, lmin, dist_ref[...])
            idx_ref[...] = jnp.where(better, larg, idx_ref[...])

        @pl.when(k == _NK - 1)
        def _():
            dist_ref[...] = jnp.sqrt(
                jnp.maximum(dist_ref[...] + lsq_acc[...], 0.0))


def _nearest(latents_flat, codebook_flat):
    out = pl.pallas_call(
        _dist_body,
        grid=(_ND, _NK),
        in_specs=[
            pl.BlockSpec((_B, _DBLK), lambda d, k: (0, d)),
            pl.BlockSpec((_KBLK, _DBLK), lambda d, k: (k, d)),
        ],
        out_specs=[
            pl.BlockSpec((_B, 1), lambda d, k: (0, 0)),
            pl.BlockSpec((_B, 1), lambda d, k: (0, 0)),
        ],
        out_shape=[
            jax.ShapeDtypeStruct((_B, 1), jnp.int32),
            jax.ShapeDtypeStruct((_B, 1), jnp.float32),
        ],
        scratch_shapes=[
            pltpu.VMEM((_NK, _B, _KBLK), jnp.float32),
            pltpu.VMEM((_B, 1), jnp.float32),
        ],
    )(latents_flat, codebook_flat)
    return out[0][:, 0], out[1][:, 0]


def _sc_gather(codebook_flat, indices):
    info = plsc.get_sparse_core_info()
    nw = info.num_cores * info.num_subcores
    bpw = _B // nw
    idx2d = indices.reshape(nw, bpw)
    mesh = plsc.VectorSubcoreMesh(core_axis_name="c", subcore_axis_name="s")

    @functools.partial(
        pl.kernel,
        mesh=mesh,
        out_type=jax.ShapeDtypeStruct((_B, _D), jnp.float32),
        scratch_types=[
            pltpu.VMEM((bpw,), jnp.int32),
            pltpu.VMEM((bpw, _D), jnp.float32),
            pltpu.SemaphoreType.DMA,
        ],
    )
    def gather(table_hbm, idx_hbm, out_hbm, idx_v, rows_v, sem):
        wid = lax.axis_index("s") * info.num_cores + lax.axis_index("c")
        pltpu.sync_copy(idx_hbm.at[wid], idx_v)
        pltpu.async_copy(table_hbm.at[idx_v], rows_v, sem).wait()
        pltpu.sync_copy(rows_v, out_hbm.at[pl.ds(wid * bpw, bpw)])

    return gather(codebook_flat, idx2d)


def kernel(latents, codebook):
    latents_flat = latents.reshape(_B, _D)
    codebook_flat = codebook.reshape(_K, _D)
    indices, min_distances = _nearest(latents_flat, codebook_flat)
    quantized = _sc_gather(codebook_flat, indices)
    quantized = quantized.reshape((_B,) + codebook.shape[1:])
    return (indices, quantized, min_distances)
